# Initial kernel scaffold; baseline (speedup 1.0000x reference)
import functools

import jax
import jax.numpy as jnp
from jax import lax
from jax.experimental import pallas as pl
from jax.experimental.pallas import tpu as pltpu
from jax.experimental.pallas import tpu_sc as plsc

N = 10000
E = 160000
E_PAD = 163840
D_T = 272
BE = 1024
BN = 1000
GW = 64
CH = 128
SIGMAS = [1.5 ** i for i in range(15)]


def _lrelu(t):
    return jnp.where(t >= 0, t, 0.01 * t)


def _ln(t, g, b):
    m = jnp.mean(t, axis=-1, keepdims=True)
    v = jnp.mean((t - m) ** 2, axis=-1, keepdims=True)
    return (t - m) * jax.lax.rsqrt(v + 1e-5) * g + b


def _dot(a, b):
    return jnp.dot(a, b, preferred_element_type=jnp.float32)


def _prep_body(h_ref, hi_ref, xp_ref, w1a_ref, w1b_ref, eb1_ref, ng_ref,
               nb_ref, wa_ref, wc_ref, wd_ref, nb1_ref, t1_ref, t2_ref,
               c_ref):
    h = h_ref[...]
    t1_ref[:, 0:256] = _dot(h, w1a_ref[...]) + eb1_ref[...]
    t1_ref[:, 256:272] = xp_ref[...]
    t2_ref[:, 0:256] = _dot(h, w1b_ref[...])
    t2_ref[:, 256:272] = xp_ref[...]
    hn = _ln(h, ng_ref[...], nb_ref[...])
    c_ref[...] = (_dot(hn, wa_ref[...]) + _dot(h, wc_ref[...])
                  + _dot(hi_ref[...], wd_ref[...]) + nb1_ref[...])


def _prep_call(h, h_init, xpad, w1a, w1b, eb1r, ngr, nbr, wa, wc, wd, nb1r):
    row = lambda i: (i, 0)
    full = lambda i: (0, 0)
    return pl.pallas_call(
        _prep_body,
        grid=(N // BN,),
        in_specs=[
            pl.BlockSpec((BN, 256), row),
            pl.BlockSpec((BN, 256), row),
            pl.BlockSpec((BN, 16), row),
            pl.BlockSpec((256, 256), full),
            pl.BlockSpec((256, 256), full),
            pl.BlockSpec((1, 256), full),
            pl.BlockSpec((1, 256), full),
            pl.BlockSpec((1, 256), full),
            pl.BlockSpec((256, 256), full),
            pl.BlockSpec((256, 256), full),
            pl.BlockSpec((256, 256), full),
            pl.BlockSpec((1, 256), full),
        ],
        out_specs=[
            pl.BlockSpec((BN, D_T), row),
            pl.BlockSpec((BN, D_T), row),
            pl.BlockSpec((BN, 256), row),
        ],
        out_shape=[
            jax.ShapeDtypeStruct((N, D_T), jnp.float32),
            jax.ShapeDtypeStruct((N, D_T), jnp.float32),
            jax.ShapeDtypeStruct((N, 256), jnp.float32),
        ],
    )(h, h_init, xpad, w1a, w1b, eb1r, ngr, nbr, wa, wc, wd, nb1r)


def _sc_gather(t1, t2, srcp, dstp):
    mesh = plsc.VectorSubcoreMesh(core_axis_name="c", subcore_axis_name="s")

    @functools.partial(
        pl.kernel,
        out_type=[
            jax.ShapeDtypeStruct((E_PAD, D_T), jnp.float32),
            jax.ShapeDtypeStruct((E_PAD, D_T), jnp.float32),
        ],
        mesh=mesh,
    )
    def k(t1_hbm, t2_hbm, s_hbm, d_hbm, g1_hbm, g2_hbm):
        def body(i1, i2, o1, o2):
            pltpu.sync_copy(t1_hbm.at[i1.at[0]], o1)
            pltpu.sync_copy(t2_hbm.at[i2.at[0]], o2)

        pltpu.emit_pipeline(
            body,
            grid=(E_PAD // GW,),
            in_specs=[
                pl.BlockSpec((1, GW), lambda i: (0, i)),
                pl.BlockSpec((1, GW), lambda i: (0, i)),
            ],
            out_specs=[
                pl.BlockSpec((GW, D_T), lambda i: (i, 0)),
                pl.BlockSpec((GW, D_T), lambda i: (i, 0)),
            ],
            core_axis_name=("c", "s"),
            dimension_semantics=(pltpu.PARALLEL,),
        )(s_hbm, d_hbm, g1_hbm, g2_hbm)

    return k(t1, t2, srcp, dstp)


def _edge_body(g1_ref, g2_ref, ea_ref, w1c_ref, w1d_ref, nis_ref, ew2_ref,
               eb2_ref, eg1_ref, ebt1_ref, cw1_ref, cb1_ref, cw2_ref,
               m_ref, t16_ref):
    g1 = g1_ref[...]
    g2 = g2_ref[...]
    a = g1[:, 0:256]
    xs = g1[:, 256:272]
    b = g2[:, 0:256]
    xd = g2[:, 256:272]
    xr = xs - xd
    d2r = jnp.sum(xr * xr, axis=1, keepdims=True)
    nt = jnp.sqrt(d2r) + 1.0
    xr = xr / nt
    d2 = d2r / (nt * nt)
    mag = jnp.exp(d2 * nis_ref[...])
    z = a + b + _dot(ea_ref[...], w1c_ref[...]) + _dot(mag, w1d_ref[...])
    m1 = _ln(_lrelu(z), eg1_ref[...], ebt1_ref[...])
    m = _lrelu(_dot(m1, ew2_ref[...]) + eb2_ref[...])
    cwh = _lrelu(_dot(m, cw1_ref[...]) + cb1_ref[...])
    cw = jnp.sum(cwh * cw2_ref[...], axis=1, keepdims=True)
    eid = jax.lax.broadcasted_iota(jnp.int32, (BE, 1), 0) + pl.program_id(0) * BE
    valid = eid < E
    m_ref[...] = jnp.where(valid, m, 0.0)
    tr = xr * cw
    lane = jax.lax.broadcasted_iota(jnp.int32, (BE, 16), 1)
    tr = jnp.where(lane == 3, 1.0, tr)
    t16_ref[...] = jnp.where(valid, tr, 0.0)


def _edge_call(g1, g2, ea_pad, w1c, w1d, nis, ew2, eb2r, eg1r, ebt1r, cw1,
               cb1r, cw2r):
    row = lambda i: (i, 0)
    full = lambda i: (0, 0)
    return pl.pallas_call(
        _edge_body,
        grid=(E_PAD // BE,),
        in_specs=[
            pl.BlockSpec((BE, D_T), row),
            pl.BlockSpec((BE, D_T), row),
            pl.BlockSpec((BE, 16), row),
            pl.BlockSpec((16, 256), full),
            pl.BlockSpec((16, 256), full),
            pl.BlockSpec((1, 16), full),
            pl.BlockSpec((256, 256), full),
            pl.BlockSpec((1, 256), full),
            pl.BlockSpec((1, 256), full),
            pl.BlockSpec((1, 256), full),
            pl.BlockSpec((256, 256), full),
            pl.BlockSpec((1, 256), full),
            pl.BlockSpec((1, 256), full),
        ],
        out_specs=[
            pl.BlockSpec((BE, 256), row),
            pl.BlockSpec((BE, 16), row),
        ],
        out_shape=[
            jax.ShapeDtypeStruct((E_PAD, 256), jnp.float32),
            jax.ShapeDtypeStruct((E_PAD, 16), jnp.float32),
        ],
    )(g1, g2, ea_pad, w1c, w1d, nis, ew2, eb2r, eg1r, ebt1r, cw1, cb1r, cw2r)


def _sc_scatter(m, t16, dst2, zeros):
    mesh = plsc.VectorSubcoreMesh(core_axis_name="c", subcore_axis_name="s")
    m_chunks = E_PAD // 16 // CH
    t_chunks = E_PAD // 2 // 16 // CH
    rows = N // 16

    @functools.partial(
        pl.kernel,
        out_type=[
            jax.ShapeDtypeStruct((N, 256), jnp.float32),
            jax.ShapeDtypeStruct((2, N, 16), jnp.float32),
        ],
        mesh=mesh,
        scratch_types=[
            pltpu.VMEM((1, CH), jnp.int32),
            pltpu.VMEM((CH, 128), jnp.float32),
            pltpu.VMEM((1, CH), jnp.int32),
            pltpu.VMEM((CH, 16), jnp.float32),
            pltpu.VMEM_SHARED((N, 128), jnp.float32),
            pltpu.VMEM_SHARED((N, 16), jnp.float32),
        ],
    )
    def k(m_hbm, t_hbm, d2_hbm, z_hbm, s_hbm, tp_hbm, idx_m, mbuf, idx_t,
          tbuf, sh_m, sh_t):
        cid = lax.axis_index("c")
        sid = lax.axis_index("s")
        r0 = sid * rows
        pltpu.sync_copy(z_hbm.at[pl.ds(r0, rows)], sh_m.at[pl.ds(r0, rows)])
        pltpu.sync_copy(z_hbm.at[pl.ds(r0, rows), pl.ds(0, 16)],
                        sh_t.at[pl.ds(r0, rows)])
        plsc.subcore_barrier()

        base_m = sid * m_chunks

        @pl.loop(0, m_chunks)
        def _(c):
            ch = base_m + c
            pltpu.sync_copy(d2_hbm.at[pl.ds(ch, 1)], idx_m)
            pltpu.sync_copy(
                m_hbm.at[pl.ds(ch * CH, CH), pl.ds(cid * 128, 128)], mbuf)
            pltpu.sync_copy(mbuf, sh_m.at[idx_m.at[0]], add=True)

        base_t = cid * (E_PAD // 2 // CH) + sid * t_chunks

        @pl.loop(0, t_chunks)
        def _(c):
            ch = base_t + c
            pltpu.sync_copy(d2_hbm.at[pl.ds(ch, 1)], idx_t)
            pltpu.sync_copy(t_hbm.at[pl.ds(ch * CH, CH)], tbuf)
            pltpu.sync_copy(tbuf, sh_t.at[idx_t.at[0]], add=True)

        plsc.subcore_barrier()
        pltpu.sync_copy(sh_m.at[pl.ds(r0, rows)],
                        s_hbm.at[pl.ds(r0, rows), pl.ds(cid * 128, 128)])
        pltpu.sync_copy(sh_t.at[pl.ds(r0, rows)],
                        tp_hbm.at[cid, pl.ds(r0, rows)])

    return k(m, t16, dst2, zeros)


def _node_body(s_ref, c_ref, h_ref, x_ref, tp_ref, wb_ref, nw2_ref, ng1_ref,
               nbt1_ref, nb2_ref, hn_ref, xn_ref):
    nm = _lrelu(_dot(s_ref[...], wb_ref[...]) + c_ref[...])
    nm = _ln(nm, ng1_ref[...], nbt1_ref[...])
    hu = _dot(nm, nw2_ref[...]) + nb2_ref[...]
    hn_ref[...] = 0.75 * hu + 0.25 * h_ref[...]
    t = tp_ref[0] + tp_ref[1]
    cnt = jnp.maximum(t[:, 3:4], 1.0)
    xn_ref[...] = x_ref[...] + t[:, 0:3] / cnt


def _node_call(s, c, h, x, tp, wb, nw2, ng1r, nbt1r, nb2r):
    row = lambda i: (i, 0)
    full = lambda i: (0, 0)
    return pl.pallas_call(
        _node_body,
        grid=(N // BN,),
        in_specs=[
            pl.BlockSpec((BN, 256), row),
            pl.BlockSpec((BN, 256), row),
            pl.BlockSpec((BN, 256), row),
            pl.BlockSpec((BN, 3), row),
            pl.BlockSpec((2, BN, 16), lambda i: (0, i, 0)),
            pl.BlockSpec((256, 256), full),
            pl.BlockSpec((256, 256), full),
            pl.BlockSpec((1, 256), full),
            pl.BlockSpec((1, 256), full),
            pl.BlockSpec((1, 256), full),
        ],
        out_specs=[
            pl.BlockSpec((BN, 256), row),
            pl.BlockSpec((BN, 3), row),
        ],
        out_shape=[
            jax.ShapeDtypeStruct((N, 256), jnp.float32),
            jax.ShapeDtypeStruct((N, 3), jnp.float32),
        ],
    )(s, c, h, x, tp, wb, nw2, ng1r, nbt1r, nb2r)


def kernel(x, h, h_init, edge_attr, edge_index, ew1, eb1, eg1, ebt1, ew2,
           eb2, ng, nb, nw1, nb1, ng1, nbt1, nw2, nb2, cw1, cb1, cw2):
    f32 = jnp.float32
    xpad = jnp.pad(x, ((0, 0), (0, 13)))
    w1a = ew1[0:256]
    w1b = ew1[256:512]
    w1c = ew1[512:528]
    w1d = jnp.pad(ew1[528:543], ((0, 1), (0, 0)))
    wa = nw1[0:256]
    wb = nw1[256:512]
    wc = nw1[512:768]
    wd = nw1[768:1024]
    nis = jnp.pad(-1.0 / jnp.array(SIGMAS, f32), (0, 1)).reshape(1, 16)
    r = lambda v: v.reshape(1, 256)

    ei = jnp.concatenate(
        [edge_index, jnp.zeros((2, E_PAD - E), jnp.int32)], axis=1)
    srcp = ei[0:1]
    dstp = ei[1:2]
    dst2 = ei[1].reshape(E_PAD // CH, CH)
    ea_pad = jnp.pad(edge_attr, ((0, E_PAD - E), (0, 0)))
    zeros = jnp.zeros((N, 128), f32)

    t1, t2, c = _prep_call(h, h_init, xpad, w1a, w1b, r(eb1), r(ng), r(nb),
                           wa, wc, wd, r(nb1))
    g1, g2 = _sc_gather(t1, t2, srcp, dstp)
    m, t16 = _edge_call(g1, g2, ea_pad, w1c, w1d, nis, ew2, r(eb2), r(eg1),
                        r(ebt1), cw1, r(cb1), cw2.reshape(1, 256))
    s, tp = _sc_scatter(m, t16, dst2, zeros)
    h_new, x_new = _node_call(s, c, h, x, tp, wb, nw2, r(ng1), r(nbt1),
                              r(nb2))
    return h_new, x_new


# speedup vs baseline: 2.9314x; 2.9314x over previous
import functools

import jax
import jax.numpy as jnp
from jax import lax
from jax.experimental import pallas as pl
from jax.experimental.pallas import tpu as pltpu
from jax.experimental.pallas import tpu_sc as plsc

N = 10000
E = 160000
E_PAD = 163840
D_T = 384
BE = 1024
BN = 1000
GW = 128
CH = 128
NP = 10240
SIGMAS = [1.5 ** i for i in range(15)]


def _lrelu(t):
    return jnp.where(t >= 0, t, 0.01 * t)


def _ln(t, g, b):
    m = jnp.mean(t, axis=-1, keepdims=True)
    v = jnp.mean((t - m) ** 2, axis=-1, keepdims=True)
    return (t - m) * jax.lax.rsqrt(v + 1e-5) * g + b


def _dot(a, b):
    return jnp.dot(a, b, preferred_element_type=jnp.float32)


def _prep_body(h_ref, hi_ref, xp_ref, w1a_ref, w1b_ref, eb1_ref, ng_ref,
               nb_ref, wa_ref, wc_ref, wd_ref, nb1_ref, t1_ref, t2_ref,
               c_ref):
    h = h_ref[...]
    xp = xp_ref[...]
    zpad = jnp.zeros((xp.shape[0], D_T - 272), jnp.float32)
    t1_ref[...] = jnp.concatenate(
        [_dot(h, w1a_ref[...]) + eb1_ref[...], xp, zpad], axis=1)
    t2_ref[...] = jnp.concatenate([_dot(h, w1b_ref[...]), xp, zpad], axis=1)
    hn = _ln(h, ng_ref[...], nb_ref[...])
    c_ref[...] = (_dot(hn, wa_ref[...]) + _dot(h, wc_ref[...])
                  + _dot(hi_ref[...], wd_ref[...]) + nb1_ref[...])


def _prep_call(h, h_init, xpad, w1a, w1b, eb1r, ngr, nbr, wa, wc, wd, nb1r):
    row = lambda i: (i, 0)
    full = lambda i: (0, 0)
    return pl.pallas_call(
        _prep_body,
        grid=(N // BN,),
        in_specs=[
            pl.BlockSpec((BN, 256), row),
            pl.BlockSpec((BN, 256), row),
            pl.BlockSpec((BN, 16), row),
            pl.BlockSpec((256, 256), full),
            pl.BlockSpec((256, 256), full),
            pl.BlockSpec((1, 256), full),
            pl.BlockSpec((1, 256), full),
            pl.BlockSpec((1, 256), full),
            pl.BlockSpec((256, 256), full),
            pl.BlockSpec((256, 256), full),
            pl.BlockSpec((256, 256), full),
            pl.BlockSpec((1, 256), full),
        ],
        out_specs=[
            pl.BlockSpec((BN, D_T), row),
            pl.BlockSpec((BN, D_T), row),
            pl.BlockSpec((BN, 256), row),
        ],
        out_shape=[
            jax.ShapeDtypeStruct((N, D_T), jnp.float32),
            jax.ShapeDtypeStruct((N, D_T), jnp.float32),
            jax.ShapeDtypeStruct((N, 256), jnp.float32),
        ],
    )(h, h_init, xpad, w1a, w1b, eb1r, ngr, nbr, wa, wc, wd, nb1r)


def _sc_gather_one(table, idx):
    mesh = plsc.VectorSubcoreMesh(core_axis_name="c", subcore_axis_name="s")

    @functools.partial(
        pl.kernel,
        out_type=jax.ShapeDtypeStruct((E_PAD, D_T), jnp.float32),
        mesh=mesh,
    )
    def k(t_hbm, i_hbm, g_hbm):
        def body(i1, o1):
            pltpu.sync_copy(t_hbm.at[i1.at[0]], o1)

        pltpu.emit_pipeline(
            body,
            grid=(E_PAD // GW,),
            in_specs=[pl.BlockSpec((1, GW), lambda i: (0, i))],
            out_specs=[pl.BlockSpec((GW, D_T), lambda i: (i, 0))],
            core_axis_name=("c", "s"),
            dimension_semantics=(pltpu.PARALLEL,),
        )(i_hbm, g_hbm)

    return k(table, idx)


def _sc_gather(t1, t2, srcp, dstp):
    return _sc_gather_one(t1, srcp), _sc_gather_one(t2, dstp)


def _edge_body(g1_ref, g2_ref, ea_ref, w1c_ref, w1d_ref, nis_ref, ew2_ref,
               eb2_ref, eg1_ref, ebt1_ref, cw1_ref, cb1_ref, cw2_ref,
               mlo_ref, mhi_ref, t16_ref):
    g1 = g1_ref[...]
    g2 = g2_ref[...]
    a = g1[:, 0:256]
    xs = g1[:, 256:272]
    b = g2[:, 0:256]
    xd = g2[:, 256:272]
    xr = xs - xd
    d2r = jnp.sum(xr * xr, axis=1, keepdims=True)
    nt = jnp.sqrt(d2r) + 1.0
    xr = xr / nt
    d2 = d2r / (nt * nt)
    mag = jnp.exp(d2 * nis_ref[...])
    z = a + b + _dot(ea_ref[...], w1c_ref[...]) + _dot(mag, w1d_ref[...])
    m1 = _ln(_lrelu(z), eg1_ref[...], ebt1_ref[...])
    m = _lrelu(_dot(m1, ew2_ref[...]) + eb2_ref[...])
    cwh = _lrelu(_dot(m, cw1_ref[...]) + cb1_ref[...])
    cw = jnp.sum(cwh * cw2_ref[...], axis=1, keepdims=True)
    eid = jax.lax.broadcasted_iota(jnp.int32, (BE, 1), 0) + pl.program_id(0) * BE
    valid = eid < E
    mv = jnp.where(valid, m, 0.0)
    mlo_ref[...] = mv[:, 0:128]
    mhi_ref[...] = mv[:, 128:256]
    tr = xr * cw
    lane = jax.lax.broadcasted_iota(jnp.int32, (BE, 16), 1)
    tr = jnp.where(lane == 3, 1.0, tr)
    t16_ref[...] = jnp.where(valid, tr, 0.0)


def _edge_call(g1, g2, ea_pad, w1c, w1d, nis, ew2, eb2r, eg1r, ebt1r, cw1,
               cb1r, cw2r):
    row = lambda i: (i, 0)
    full = lambda i: (0, 0)
    return pl.pallas_call(
        _edge_body,
        grid=(E_PAD // BE,),
        in_specs=[
            pl.BlockSpec((BE, D_T), row),
            pl.BlockSpec((BE, D_T), row),
            pl.BlockSpec((BE, 16), row),
            pl.BlockSpec((16, 256), full),
            pl.BlockSpec((16, 256), full),
            pl.BlockSpec((1, 16), full),
            pl.BlockSpec((256, 256), full),
            pl.BlockSpec((1, 256), full),
            pl.BlockSpec((1, 256), full),
            pl.BlockSpec((1, 256), full),
            pl.BlockSpec((256, 256), full),
            pl.BlockSpec((1, 256), full),
            pl.BlockSpec((1, 256), full),
        ],
        out_specs=[
            pl.BlockSpec((BE, 128), row),
            pl.BlockSpec((BE, 128), row),
            pl.BlockSpec((BE, 16), row),
        ],
        out_shape=[
            jax.ShapeDtypeStruct((E_PAD, 128), jnp.float32),
            jax.ShapeDtypeStruct((E_PAD, 128), jnp.float32),
            jax.ShapeDtypeStruct((E_PAD, 16), jnp.float32),
        ],
    )(g1, g2, ea_pad, w1c, w1d, nis, ew2, eb2r, eg1r, ebt1r, cw1, cb1r, cw2r)


GR = 5120
SHM_ROWS = 5248


def _sc_scatter_m(m_lo, m_hi, dst1, zeros):
    mesh = plsc.VectorSubcoreMesh(core_axis_name="c", subcore_axis_name="s")
    m_chunks = E_PAD // 16 // CH
    zrows = SHM_ROWS // 16
    wrows = GR // 16

    @functools.partial(
        pl.kernel,
        out_type=jax.ShapeDtypeStruct((2, NP, 128), jnp.float32),
        mesh=mesh,
        scratch_types=[
            pltpu.VMEM((CH,), jnp.int32),
            pltpu.VMEM((CH,), jnp.int32),
            pltpu.VMEM((CH, 128), jnp.float32),
            pltpu.VMEM_SHARED((SHM_ROWS, 128), jnp.float32),
        ],
    )
    def k(mlo_hbm, mhi_hbm, d_hbm, z_hbm, s2_hbm, idx_raw, idx2, mbuf, sh_m):
        cid = lax.axis_index("c")
        sid = lax.axis_index("s")
        base_m = sid * m_chunks

        for p in range(2):
            lo = p * GR
            z0 = sid * zrows
            pltpu.sync_copy(z_hbm.at[pl.ds(z0, zrows)],
                            sh_m.at[pl.ds(z0, zrows)])
            plsc.subcore_barrier()

            def m_scan(m_hbm):
                @pl.loop(0, m_chunks)
                def _(c):
                    e0 = (base_m + c) * CH
                    pltpu.sync_copy(d_hbm.at[pl.ds(e0, CH)], idx_raw)
                    pltpu.sync_copy(m_hbm.at[pl.ds(e0, CH)], mbuf)
                    for j in range(CH // 16):
                        v = idx_raw[pl.ds(j * 16, 16)] - lo
                        ok = (v >= 0) & (v < GR)
                        idx2[pl.ds(j * 16, 16)] = jnp.where(ok, v, GR)
                    pltpu.sync_copy(mbuf, sh_m.at[idx2], add=True)

            @pl.when(cid == 0)
            def _():
                m_scan(mlo_hbm)

            @pl.when(cid == 1)
            def _():
                m_scan(mhi_hbm)

            plsc.subcore_barrier()
            w0 = sid * wrows
            pltpu.sync_copy(sh_m.at[pl.ds(w0, wrows)],
                            s2_hbm.at[cid].at[pl.ds(lo + w0, wrows)])
            plsc.subcore_barrier()

    return k(m_lo, m_hi, dst1, zeros)


def _sc_scatter_t(t16, dst1, zeros16):
    mesh = plsc.VectorSubcoreMesh(core_axis_name="c", subcore_axis_name="s")
    t_chunks = E_PAD // 2 // 16 // CH
    trows = NP // 16

    @functools.partial(
        pl.kernel,
        out_type=jax.ShapeDtypeStruct((2, NP, 16), jnp.float32),
        mesh=mesh,
        scratch_types=[
            pltpu.VMEM((CH,), jnp.int32),
            pltpu.VMEM((CH, 16), jnp.float32),
            pltpu.VMEM_SHARED((NP, 16), jnp.float32),
        ],
    )
    def k(t_hbm, d_hbm, z16_hbm, tp_hbm, idx_raw, tbuf, sh_t):
        cid = lax.axis_index("c")
        sid = lax.axis_index("s")
        base_t = cid * (E_PAD // 2 // CH) + sid * t_chunks
        t0 = sid * trows
        pltpu.sync_copy(z16_hbm.at[pl.ds(t0, trows)], sh_t.at[pl.ds(t0, trows)])
        plsc.subcore_barrier()

        @pl.loop(0, t_chunks)
        def _(c):
            e0 = (base_t + c) * CH
            pltpu.sync_copy(d_hbm.at[pl.ds(e0, CH)], idx_raw)
            pltpu.sync_copy(t_hbm.at[pl.ds(e0, CH)], tbuf)
            pltpu.sync_copy(tbuf, sh_t.at[idx_raw], add=True)

        plsc.subcore_barrier()
        pltpu.sync_copy(sh_t.at[pl.ds(t0, trows)],
                        tp_hbm.at[cid].at[pl.ds(t0, trows)])

    return k(t16, dst1, zeros16)


def _sc_scatter(m_lo, m_hi, t16, dst1, zeros, zeros16):
    return (_sc_scatter_m(m_lo, m_hi, dst1, zeros),
            _sc_scatter_t(t16, dst1, zeros16))


def _node_body(s_ref, c_ref, h_ref, x_ref, tp_ref, wb_ref, nw2_ref, ng1_ref,
               nbt1_ref, nb2_ref, hn_ref, xn_ref):
    agg = _dot(s_ref[0], wb_ref[0:128, :]) + _dot(s_ref[1], wb_ref[128:256, :])
    nm = _lrelu(agg + c_ref[...])
    nm = _ln(nm, ng1_ref[...], nbt1_ref[...])
    hu = _dot(nm, nw2_ref[...]) + nb2_ref[...]
    hn_ref[...] = 0.75 * hu + 0.25 * h_ref[...]
    t = tp_ref[0] + tp_ref[1]
    cnt = jnp.maximum(t[:, 3:4], 1.0)
    xn_ref[...] = x_ref[...] + t[:, 0:3] / cnt


def _node_call(s, c, h, x, tp, wb, nw2, ng1r, nbt1r, nb2r):
    row = lambda i: (i, 0)
    full = lambda i: (0, 0)
    return pl.pallas_call(
        _node_body,
        grid=(N // BN,),
        in_specs=[
            pl.BlockSpec((2, BN, 128), lambda i: (0, i, 0)),
            pl.BlockSpec((BN, 256), row),
            pl.BlockSpec((BN, 256), row),
            pl.BlockSpec((BN, 3), row),
            pl.BlockSpec((2, BN, 16), lambda i: (0, i, 0)),
            pl.BlockSpec((256, 256), full),
            pl.BlockSpec((256, 256), full),
            pl.BlockSpec((1, 256), full),
            pl.BlockSpec((1, 256), full),
            pl.BlockSpec((1, 256), full),
        ],
        out_specs=[
            pl.BlockSpec((BN, 256), row),
            pl.BlockSpec((BN, 3), row),
        ],
        out_shape=[
            jax.ShapeDtypeStruct((N, 256), jnp.float32),
            jax.ShapeDtypeStruct((N, 3), jnp.float32),
        ],
    )(s, c, h, x, tp, wb, nw2, ng1r, nbt1r, nb2r)


def kernel(x, h, h_init, edge_attr, edge_index, ew1, eb1, eg1, ebt1, ew2,
           eb2, ng, nb, nw1, nb1, ng1, nbt1, nw2, nb2, cw1, cb1, cw2):
    f32 = jnp.float32
    xpad = jnp.pad(x, ((0, 0), (0, 13)))
    w1a = ew1[0:256]
    w1b = ew1[256:512]
    w1c = ew1[512:528]
    w1d = jnp.pad(ew1[528:543], ((0, 1), (0, 0)))
    wa = nw1[0:256]
    wb = nw1[256:512]
    wc = nw1[512:768]
    wd = nw1[768:1024]
    nis = jnp.pad(-1.0 / jnp.array(SIGMAS, f32), (0, 1)).reshape(1, 16)
    r = lambda v: v.reshape(1, 256)

    ei = jnp.concatenate(
        [edge_index, jnp.zeros((2, E_PAD - E), jnp.int32)], axis=1)
    srcp = ei[0:1]
    dstp = ei[1:2]
    dst1 = ei[1]
    ea_pad = jnp.pad(edge_attr, ((0, E_PAD - E), (0, 0)))
    zeros = jnp.zeros((NP, 128), f32)
    zeros16 = jnp.zeros((NP, 16), f32)

    t1, t2, c = _prep_call(h, h_init, xpad, w1a, w1b, r(eb1), r(ng), r(nb),
                           wa, wc, wd, r(nb1))
    g1, g2 = _sc_gather(t1, t2, srcp, dstp)
    m_lo, m_hi, t16 = _edge_call(g1, g2, ea_pad, w1c, w1d, nis, ew2, r(eb2),
                                 r(eg1), r(ebt1), cw1, r(cb1),
                                 cw2.reshape(1, 256))
    s, tp = _sc_scatter(m_lo, m_hi, t16, dst1, zeros, zeros16)
    h_new, x_new = _node_call(s, c, h, x, tp, wb, nw2, r(ng1), r(nbt1),
                              r(nb2))
    return h_new, x_new
